# baseline (speedup 1.0000x reference)
import functools

import jax
import jax.numpy as jnp
from jax import lax
from jax.experimental import pallas as pl
from jax.experimental.pallas import tpu as pltpu
from jax.experimental.pallas import tpu_sc as plsc

B = 4096
P = 36864
PT = 1024
NP = P // PT
WIN = 128
NWT = PT // WIN
RB = 64
NSTEP = B // RB

SC_CORES = 2
SC_SUBCORES = 16
NW = SC_CORES * SC_SUBCORES
B_PER_W = B // NW


def _k1_body(a_ref, t_ref, best_ref, w_ref):
    j = pl.program_id(0)
    s = jnp.dot(a_ref[...], t_ref[...], preferred_element_type=jnp.float32)

    @pl.when(j == 0)
    def _():
        best_ref[...] = jnp.full((B,), -jnp.inf, jnp.float32)
        w_ref[...] = jnp.zeros((B,), jnp.int32)

    best = best_ref[...]
    wid = w_ref[...]
    for c in range(NWT):
        mc = jnp.max(s[:, c * WIN:(c + 1) * WIN], axis=1)
        upd = mc > best
        best = jnp.where(upd, mc, best)
        wid = jnp.where(upd, j * NWT + c, wid)
    best_ref[...] = best
    w_ref[...] = wid


def _k2_body(w_sref, a_ref, best_ref, wv_ref, *rest):
    t_refs = rest[:RB]
    idx_ref = rest[RB]
    a_blk = a_ref[...]
    tstack = jnp.concatenate([t_refs[r][...] for r in range(RB)], axis=0)
    s_full = lax.dot_general(
        a_blk, tstack[:, :9], (((1,), (1,)), ((), ())),
        preferred_element_type=jnp.float32)
    best_blk = best_ref[...]
    iota = lax.broadcasted_iota(jnp.int32, (1, WIN), 1)
    cols = []
    for r in range(RB):
        s_r = s_full[r:r + 1, r * WIN:(r + 1) * WIN]
        b_r = best_blk[0, 0:1, r:r + 1]
        eq = s_r == b_r
        c1 = jnp.min(jnp.where(eq, iota, WIN), axis=1, keepdims=True)
        m_r = jnp.max(s_r, axis=1, keepdims=True)
        c2 = jnp.min(jnp.where(s_r == m_r, iota, WIN - 1), axis=1, keepdims=True)
        cols.append(jnp.where(c1 < WIN, c1, c2))
    cvec = jnp.concatenate(cols, axis=1)
    idx_ref[...] = wv_ref[...] * WIN + cvec.reshape(1, 1, RB)


def _sc_gather(table_pad, idxs):
    mesh = plsc.VectorSubcoreMesh(core_axis_name="c", subcore_axis_name="s")

    @functools.partial(
        pl.kernel,
        mesh=mesh,
        out_type=jax.ShapeDtypeStruct((B, 16), jnp.float32),
        scratch_types=[
            pltpu.VMEM((B_PER_W,), jnp.int32),
            pltpu.VMEM((B_PER_W, 16), jnp.float32),
            pltpu.SemaphoreType.DMA,
        ],
        compiler_params=pltpu.CompilerParams(use_tc_tiling_on_sc=False),
    )
    def gather_k(table_hbm, idx_hbm, out_hbm, idx_v, rows_v, sem):
        wid = lax.axis_index("s") * SC_CORES + lax.axis_index("c")
        base = wid * B_PER_W
        pltpu.sync_copy(idx_hbm.at[pl.ds(base, B_PER_W)], idx_v)
        pltpu.async_copy(table_hbm.at[idx_v], rows_v, sem).wait()
        pltpu.sync_copy(rows_v, out_hbm.at[pl.ds(base, B_PER_W)])

    return gather_k(table_pad, idxs)


def kernel(rotMat, output_rotmats):
    a = rotMat.reshape(B, 9)
    t = output_rotmats.reshape(P, 9)
    tt = t.T

    best, w = pl.pallas_call(
        _k1_body,
        grid=(NP,),
        in_specs=[
            pl.BlockSpec((B, 9), lambda j: (0, 0)),
            pl.BlockSpec((9, PT), lambda j: (0, j)),
        ],
        out_specs=[
            pl.BlockSpec((B,), lambda j: (0,)),
            pl.BlockSpec((B,), lambda j: (0,)),
        ],
        out_shape=[
            jax.ShapeDtypeStruct((B,), jnp.float32),
            jax.ShapeDtypeStruct((B,), jnp.int32),
        ],
    )(a, tt)

    table_pad = jnp.pad(t, ((0, 0), (0, 7)))

    def t_spec(r):
        return pl.BlockSpec((WIN, 16), lambda i, ws, r=r: (ws[i * RB + r], 0))

    idx3 = pl.pallas_call(
        _k2_body,
        grid_spec=pltpu.PrefetchScalarGridSpec(
            num_scalar_prefetch=1,
            grid=(NSTEP,),
            in_specs=[
                pl.BlockSpec((RB, 9), lambda i, ws: (i, 0)),
                pl.BlockSpec((1, 1, RB), lambda i, ws: (i, 0, 0)),
                pl.BlockSpec((1, 1, RB), lambda i, ws: (i, 0, 0)),
            ] + [t_spec(r) for r in range(RB)],
            out_specs=pl.BlockSpec((1, 1, RB), lambda i, ws: (i, 0, 0)),
        ),
        out_shape=jax.ShapeDtypeStruct((NSTEP, 1, RB), jnp.int32),
    )(w, a, best.reshape(NSTEP, 1, RB), w.reshape(NSTEP, 1, RB),
      *([table_pad] * RB))

    rows = _sc_gather(table_pad, idx3.reshape(B))
    nearest = rows[:, :9].reshape(B, 3, 3)
    return best, nearest


# speedup vs baseline: 6.2194x; 6.2194x over previous
import functools

import jax
import jax.numpy as jnp
from jax import lax
from jax.experimental import pallas as pl
from jax.experimental.pallas import tpu as pltpu
from jax.experimental.pallas import tpu_sc as plsc

B = 4096
P = 36864
PT = 2048
NP = P // PT

SC_CORES = 2
SC_SUBCORES = 16
NW = SC_CORES * SC_SUBCORES
B_PER_W = B // NW


def _argmax_body(a_ref, t_ref, iota_ref, best_ref, idx_ref):
    j = pl.program_id(0)
    s = jnp.dot(a_ref[...], t_ref[...], preferred_element_type=jnp.float32)
    m = jnp.max(s, axis=1, keepdims=True)
    iota = iota_ref[...]
    loc = jnp.min(jnp.where(s == m, iota, jnp.float32(P)), axis=1, keepdims=True)

    @pl.when(j == 0)
    def _():
        best_ref[...] = m
        idx_ref[...] = loc

    @pl.when(j > 0)
    def _():
        prev = best_ref[...]
        upd = m > prev
        best_ref[...] = jnp.where(upd, m, prev)
        idx_ref[...] = jnp.where(upd, loc, idx_ref[...])


def _sc_gather(table_pad, idxs):
    mesh = plsc.VectorSubcoreMesh(core_axis_name="c", subcore_axis_name="s")

    @functools.partial(
        pl.kernel,
        mesh=mesh,
        out_type=jax.ShapeDtypeStruct((B, 16), jnp.float32),
        scratch_types=[
            pltpu.VMEM((B_PER_W,), jnp.int32),
            pltpu.VMEM((B_PER_W, 16), jnp.float32),
            pltpu.SemaphoreType.DMA,
        ],
        compiler_params=pltpu.CompilerParams(use_tc_tiling_on_sc=False),
    )
    def gather_k(table_hbm, idx_hbm, out_hbm, idx_v, rows_v, sem):
        wid = lax.axis_index("s") * SC_CORES + lax.axis_index("c")
        base = wid * B_PER_W
        pltpu.sync_copy(idx_hbm.at[pl.ds(base, B_PER_W)], idx_v)
        pltpu.async_copy(table_hbm.at[idx_v], rows_v, sem).wait()
        pltpu.sync_copy(rows_v, out_hbm.at[pl.ds(base, B_PER_W)])

    return gather_k(table_pad, idxs)


def kernel(rotMat, output_rotmats):
    a = rotMat.reshape(B, 9)
    t = output_rotmats.reshape(P, 9)
    tt = t.T
    iota_f = jnp.arange(P, dtype=jnp.float32).reshape(1, P)

    best, idx = pl.pallas_call(
        _argmax_body,
        grid=(NP,),
        in_specs=[
            pl.BlockSpec((B, 9), lambda j: (0, 0)),
            pl.BlockSpec((9, PT), lambda j: (0, j)),
            pl.BlockSpec((1, PT), lambda j: (0, j)),
        ],
        out_specs=[
            pl.BlockSpec((B, 1), lambda j: (0, 0)),
            pl.BlockSpec((B, 1), lambda j: (0, 0)),
        ],
        out_shape=[
            jax.ShapeDtypeStruct((B, 1), jnp.float32),
            jax.ShapeDtypeStruct((B, 1), jnp.float32),
        ],
    )(a, tt, iota_f)

    table_pad = jnp.pad(t, ((0, 0), (0, 7)))
    rows = _sc_gather(table_pad, idx.reshape(B).astype(jnp.int32))
    nearest = rows[:, :9].reshape(B, 3, 3)
    return best.reshape(B), nearest
